# baseline (speedup 1.0000x reference)
import jax
import jax.numpy as jnp
from jax.experimental import pallas as pl
from jax.experimental.pallas import tpu as pltpu

_N = 1024
_T = 32
_TNA = 2
_F = 16


def _body(x_ref, d1_ref, d2_ref):
    t = pl.program_id(0)
    xb = x_ref[0, 0]
    nanf = jnp.where(xb != xb, 1.0, 0.0)
    invalid = jnp.max(nanf, axis=1, keepdims=True)
    validf = 1.0 - invalid

    ik = jax.lax.broadcasted_iota(jnp.int32, (_N, _N), 0)
    ic = jax.lax.broadcasted_iota(jnp.int32, (_N, _N), 1)
    tri = jnp.where(ik <= ic, 1.0, 0.0)
    incl = jax.lax.dot_general(
        validf, tri, (((0,), (0,)), ((), ())),
        preferred_element_type=jnp.float32)

    inclh = incl[:, 1:]
    incll = incl[:, : _N - 1]
    vs = inclh - incll
    r = inclh - 1.0
    rows = jax.lax.broadcasted_iota(
        jnp.int32, (_N, _N - 1), 0).astype(jnp.float32)
    d1_ref[0] = jnp.where((rows == r) & (vs > 0.5), 1.0, 0.0)

    c2 = jax.lax.broadcasted_iota(jnp.int32, (_N, _T - _TNA - 1), 1)
    d2_ref[0] = jnp.where(c2 == t - (_TNA + 1), 1.0, 0.0)


def kernel(x):
    d1, d2 = pl.pallas_call(
        _body,
        grid=(_T,),
        in_specs=[pl.BlockSpec((1, 1, _N, _F), lambda t: (0, t, 0, 0))],
        out_specs=[
            pl.BlockSpec((1, _N, _N - 1), lambda t: (0, t, 0)),
            pl.BlockSpec((1, _N, _T - _TNA - 1), lambda t: (0, t, 0)),
        ],
        out_shape=[
            jax.ShapeDtypeStruct((1, _T * _N, _N - 1), jnp.float32),
            jax.ShapeDtypeStruct((1, _T * _N, _T - _TNA - 1), jnp.float32),
        ],
        compiler_params=pltpu.CompilerParams(
            dimension_semantics=("parallel",)),
    )(x)
    return (d1, d2)


# speedup vs baseline: 2.0182x; 2.0182x over previous
import jax
import jax.numpy as jnp
from jax.experimental import pallas as pl
from jax.experimental.pallas import tpu as pltpu

_N = 1024
_T = 32
_TNA = 2
_F = 16


def _body(x_ref, d1_ref, d2_ref):
    t = pl.program_id(0)
    xb = x_ref[0, 0]
    nanf = jnp.where(xb != xb, 1.0, 0.0)
    invalid = jnp.max(nanf, axis=1, keepdims=True)
    validf = 1.0 - invalid

    ik = jax.lax.broadcasted_iota(jnp.int32, (_N, _N), 0)
    ic = jax.lax.broadcasted_iota(jnp.int32, (_N, _N), 1)
    tri = jnp.where(ik <= ic, 1.0, 0.0)
    incl = jax.lax.dot_general(
        validf, tri, (((0,), (0,)), ((), ())),
        preferred_element_type=jnp.float32)

    inclh = incl[:, 1:]
    incll = incl[:, : _N - 1]
    vs = inclh - incll
    r = inclh - 1.0
    rows = jax.lax.broadcasted_iota(
        jnp.int32, (_N, _N - 1), 0).astype(jnp.float32)
    d1_ref[0] = jnp.where((rows == r) & (vs > 0.5), 1.0, 0.0)

    c2 = jax.lax.broadcasted_iota(jnp.int32, (_N, _T - _TNA - 1), 1)
    d2_ref[0] = jnp.where(c2 == t - (_TNA + 1), 1.0, 0.0)


def kernel(x):
    d1, d2 = pl.pallas_call(
        _body,
        grid=(_T,),
        in_specs=[pl.BlockSpec((1, 1, _N, _F), lambda t: (0, t, 0, 0))],
        out_specs=[
            pl.BlockSpec((1, _N, _N - 1), lambda t: (t, 0, 0)),
            pl.BlockSpec((1, _N, _T - _TNA - 1), lambda t: (t, 0, 0)),
        ],
        out_shape=[
            jax.ShapeDtypeStruct((_T, _N, _N - 1), jnp.float32),
            jax.ShapeDtypeStruct((_T, _N, _T - _TNA - 1), jnp.float32),
        ],
        compiler_params=pltpu.CompilerParams(
            dimension_semantics=("parallel",)),
    )(x)
    return (d1.reshape(1, _T * _N, _N - 1), d2.reshape(1, _T * _N, _T - _TNA - 1))
